# baseline (speedup 1.0000x reference)
import functools

import jax
import jax.numpy as jnp
import numpy as np
from jax.experimental import pallas as pl
from jax.experimental.pallas import tpu as pltpu

END_BAND = 64
RING = 16
LEAD = 8


def _band_indices(num_bands):
    try:
        with jax.ensure_compile_time_eval():
            perm_key = jax.random.key(42)
            perm = jax.random.permutation(perm_key, num_bands)[:END_BAND]
            return jnp.asarray(np.asarray(perm, dtype=np.int32))
    except Exception:
        perm_key = jax.random.key(42)
        perm = jax.random.permutation(perm_key, num_bands)[:END_BAND]
        return perm.astype(jnp.int32)


def _gather_kernel(idx_ref, x_hbm, o_hbm, buf, rsem, wsem):
    def read(t):
        b = t % RING
        return pltpu.make_async_copy(x_hbm.at[idx_ref[t]], buf.at[b], rsem.at[b])

    def write(t):
        b = t % RING
        return pltpu.make_async_copy(buf.at[b], o_hbm.at[t], wsem.at[b])

    for t in range(min(LEAD, END_BAND)):
        read(t).start()
    for t in range(END_BAND):
        read(t).wait()
        write(t).start()
        tr = t + LEAD
        if tr < END_BAND:
            if tr >= RING:
                write(tr - RING).wait()
            read(tr).start()
    for t in range(max(0, END_BAND - RING), END_BAND):
        write(t).wait()


def kernel(x):
    num_bands = x.shape[0]
    if num_bands <= END_BAND:
        return x
    indices = _band_indices(num_bands)
    H, W = x.shape[1], x.shape[2]
    grid_spec = pltpu.PrefetchScalarGridSpec(
        num_scalar_prefetch=1,
        grid=(),
        in_specs=[pl.BlockSpec(memory_space=pl.ANY)],
        out_specs=pl.BlockSpec(memory_space=pl.ANY),
        scratch_shapes=[
            pltpu.VMEM((RING, H, W), jnp.float32),
            pltpu.SemaphoreType.DMA((RING,)),
            pltpu.SemaphoreType.DMA((RING,)),
        ],
    )
    return pl.pallas_call(
        _gather_kernel,
        grid_spec=grid_spec,
        out_shape=jax.ShapeDtypeStruct((END_BAND, H, W), x.dtype),
    )(indices, x)
